# baseline (speedup 1.0000x reference)
import functools

import jax
import jax.numpy as jnp
from jax import lax
from jax.experimental import pallas as pl
from jax.experimental.pallas import tpu as pltpu
from jax.experimental.pallas import tpu_sc as plsc

_NEG = -1e30
_NC = 2
_NS = 16
_NW = _NC * _NS
_L = 16
_K = 8
_TOPK = _K - 1
_B = 256


def _logits_kernel(x_ref, gw_ref, out_ref):
    out_ref[...] = lax.dot_general(
        gw_ref[...], x_ref[...], (((1,), (1,)), ((), ())),
        preferred_element_type=jnp.float32)


def _sc_route_body(lg_hbm, bias_hbm, zf_hbm, zi_hbm,
                   rw_hbm, islot_hbm, tok_hbm, wslot_hbm, be_hbm, xch_hbm,
                   lg_v, bias_v, rw_v, selT, rankT, picks_v, wv_v, posidx_v,
                   tokid_v, cnt_v, cnt_all, segb_v, segs_v, pref_v, nb_v,
                   bev_v, zbuf_f, zbuf_i,
                   *, n_exp, tpw, half_slots, nbh):
    c = lax.axis_index("c")
    s = lax.axis_index("s")
    wid = c * _NS + s
    tokbase = wid * tpw
    zslots = (half_slots * _NC) // _NW
    pltpu.sync_copy(lg_hbm.at[wid], lg_v)
    pltpu.sync_copy(bias_hbm, bias_v)
    pltpu.sync_copy(zf_hbm, rw_v)
    pltpu.sync_copy(zf_hbm.at[pl.ds(0, n_exp * tpw)], selT)
    pltpu.sync_copy(zf_hbm.at[pl.ds(0, zslots)], zbuf_f)
    pltpu.sync_copy(zi_hbm, zbuf_i)
    pltpu.sync_copy(zbuf_i, tok_hbm.at[pl.ds(wid * zslots, zslots)])
    pltpu.sync_copy(zbuf_f, wslot_hbm.at[pl.ds(wid * zslots, zslots)])
    lanes = lax.iota(jnp.int32, _L)
    ones_f = jnp.full((_L,), 1.0, jnp.float32)

    def group(g, carry):
        m = jnp.full((_L,), _NEG, jnp.float32)
        vs = []
        for r in range(1, n_exp):
            v = lg_v[g, r, :]
            vs.append(v)
            m = jnp.maximum(m, v)
        ssum = jnp.zeros((_L,), jnp.float32)
        ps = []
        for v in vs:
            p = jnp.exp(v - m)
            ps.append(p)
            ssum = ssum + p
        inv = 1.0 / ssum
        bs = [p * inv + bias_v[r + 1, :] for r, p in enumerate(ps)]
        idxs, vals = [], []
        tot = jnp.zeros((_L,), jnp.float32)
        for _ in range(_TOPK):
            mx = jnp.full((_L,), _NEG, jnp.float32)
            am = jnp.zeros((_L,), jnp.int32)
            for r in range(1, n_exp):
                cand = bs[r - 1]
                ok = cand > mx
                for prev in idxs:
                    ok = jnp.logical_and(ok, prev != r)
                mx = jnp.where(ok, cand, mx)
                am = jnp.where(ok, jnp.full((_L,), r, jnp.int32), am)
            idxs.append(am)
            vals.append(mx)
            tot = tot + mx
        scale = (_TOPK / (_TOPK + 1.0)) / tot
        tl = g * _L + lanes
        p0 = tl * _K
        c7 = jnp.full((_L,), 7, jnp.int32)
        c127 = jnp.full((_L,), 127, jnp.int32)
        plsc.store_scatter(rw_v, [tl * n_exp],
                           jnp.full((_L,), 1.0 / _K, jnp.float32))
        plsc.store_scatter(picks_v, [p0], jnp.zeros((_L,), jnp.int32))
        plsc.store_scatter(
            wv_v, [lax.shift_right_logical(p0, c7), jnp.bitwise_and(p0, c127)],
            jnp.full((_L,), 1.0 / _K, jnp.float32))
        plsc.store_scatter(selT, [tl], ones_f)
        for k in range(_TOPK):
            w = vals[k] * scale
            pk = p0 + (k + 1)
            plsc.store_scatter(rw_v, [tl * n_exp + idxs[k]], w)
            plsc.store_scatter(picks_v, [pk], idxs[k])
            plsc.store_scatter(
                wv_v,
                [lax.shift_right_logical(pk, c7), jnp.bitwise_and(pk, c127)],
                w)
            plsc.store_scatter(selT, [idxs[k] * tpw + tl], ones_f)
        return carry

    lax.fori_loop(0, tpw // _L, group, 0)

    def rank_row(e, carry):
        run = jnp.float32(0.0)
        for j in range(tpw // _L):
            ch = selT[pl.ds(e * tpw + j * _L, _L)]
            cs = plsc.cumsum(ch)
            rankT[pl.ds(e * tpw + j * _L, _L)] = cs - ch + run
            run = run + jnp.max(cs)
        return carry

    lax.fori_loop(0, n_exp, rank_row, 0)
    for q in range(n_exp // _L):
        evec = (lanes + q * _L) * tpw + (tpw - 1)
        cnt = plsc.load_gather(rankT, [evec]) + plsc.load_gather(selT, [evec])
        cnt_v[pl.ds(q * _L, _L)] = cnt.astype(jnp.int32)

    pltpu.sync_copy(cnt_v, xch_hbm.at[wid])
    plsc.subcore_barrier()
    pltpu.sync_copy(xch_hbm.at[pl.ds(c * _NS, _NS)], cnt_all)

    for q in range(n_exp // _L):
        tot = jnp.zeros((_L,), jnp.int32)
        pref = jnp.zeros((_L,), jnp.int32)
        for sp in range(_NS):
            row = cnt_all[sp, pl.ds(q * _L, _L)]
            tot = tot + row
            take = jnp.full((_L,), sp, jnp.int32) < s
            pref = pref + jnp.where(take, row, jnp.zeros((_L,), jnp.int32))
        pref_v[pl.ds(q * _L, _L)] = pref
        nb_v[pl.ds(q * _L, _L)] = lax.shift_right_logical(
            tot + (_B - 1), jnp.full((_L,), 8, jnp.int32))
    ch0 = nb_v[pl.ds(0, _L)]
    cs0 = plsc.cumsum(ch0)
    segb0 = cs0 - ch0
    carry0 = jnp.max(cs0)
    ch1 = nb_v[pl.ds(_L, _L)]
    cs1 = plsc.cumsum(ch1)
    segb1 = cs1 - ch1 + carry0
    segb_v[pl.ds(0, _L)] = segb0
    segb_v[pl.ds(_L, _L)] = segb1
    segs_v[pl.ds(0, _L)] = lax.shift_left(
        segb0, jnp.full((_L,), 8, jnp.int32))
    segs_v[pl.ds(_L, _L)] = lax.shift_left(
        segb1, jnp.full((_L,), 8, jnp.int32))

    for j in range(nbh // _L + 1):
        bev_v[pl.ds(j * _L, _L)] = jnp.zeros((_L,), jnp.int32)
    for q in range(n_exp // _L):
        segb_ch = segb_v[pl.ds(q * _L, _L)]
        nb_ch = nb_v[pl.ds(q * _L, _L)]
        for el in range(_L):
            e = q * _L + el
            idx = jnp.broadcast_to(segb_ch[el], (_L,)) + lanes
            msk = lanes < jnp.broadcast_to(nb_ch[el], (_L,))
            plsc.store_scatter(bev_v, [idx], jnp.full((_L,), e, jnp.int32),
                               mask=msk)
    pltpu.sync_copy(bev_v.at[pl.ds(0, nbh)], be_hbm.at[pl.ds(c * nbh, nbh)])

    def pair_group(p16, carry):
        p = p16 * _L + lanes
        tl = lax.shift_right_logical(p, jnp.full((_L,), 3, jnp.int32))
        e = plsc.load_gather(picks_v, [p])
        rank = plsc.load_gather(rankT, [e * tpw + tl]).astype(jnp.int32)
        seg = plsc.load_gather(segs_v, [e])
        pref = plsc.load_gather(pref_v, [e])
        pos = c * half_slots + seg + pref + rank
        prow = lax.shift_right_logical(p, jnp.full((_L,), 7, jnp.int32))
        pcol = jnp.bitwise_and(p, jnp.full((_L,), 127, jnp.int32))
        plsc.store_scatter(posidx_v, [prow, pcol], pos)
        plsc.store_scatter(tokid_v, [prow, pcol], tokbase + tl)
        return carry

    lax.fori_loop(0, (tpw * _K) // _L, pair_group, 0)

    pltpu.sync_copy(rw_v, rw_hbm.at[pl.ds(wid * tpw * n_exp, tpw * n_exp)])
    pltpu.sync_copy(posidx_v, islot_hbm.at[pl.ds(wid * (tpw * _K // 128),
                                                 tpw * _K // 128)])
    for j in range(tpw * _K // 128):
        pltpu.sync_copy(tokid_v.at[j], tok_hbm.at[posidx_v.at[j]])
        pltpu.sync_copy(wv_v.at[j], wslot_hbm.at[posidx_v.at[j]])


def _sc_gather_body(tok_hbm, x_hbm, xg_hbm, idx_v, rows_v, sem, *, rows_pw):
    c = lax.axis_index("c")
    s = lax.axis_index("s")
    wid = c * _NS + s

    def chunk(j, carry):
        row = wid * rows_pw + j
        pltpu.sync_copy(tok_hbm.at[row], idx_v)
        pltpu.async_copy(x_hbm.at[idx_v], rows_v, sem).wait()
        pltpu.sync_copy(rows_v, xg_hbm.at[pl.ds(row * 128, 128)])
        return carry

    lax.fori_loop(0, rows_pw, chunk, 0)


def _gemm_kernel(be_ref, xg_ref, w2_ref, wfc_ref, wproj_ref, yg_ref):
    x = xg_ref[...]
    h = lax.dot_general(x, wfc_ref[0], (((1,), (1,)), ((), ())),
                        preferred_element_type=jnp.float32)
    h = jnp.square(jnp.maximum(h, 0.0))
    y = lax.dot_general(h, wproj_ref[0], (((1,), (1,)), ((), ())),
                        preferred_element_type=jnp.float32)
    yg_ref[...] = y * w2_ref[0]


def _sc_combine_body(islot_hbm, yg_hbm, out_hbm, idx_v, rows_v, acc_v, sem,
                     *, d, tpw):
    c = lax.axis_index("c")
    s = lax.axis_index("s")
    wid = c * _NS + s
    tpr = 128 // _K

    def chunk(j, carry):
        isrow = wid * (tpw // tpr) + j
        pltpu.sync_copy(islot_hbm.at[isrow], idx_v)
        pltpu.async_copy(yg_hbm.at[idx_v], rows_v, sem).wait()
        for tl in range(tpr):
            for j2 in range(d // _L):
                acc = rows_v[tl * _K, pl.ds(j2 * _L, _L)]
                for k in range(1, _K):
                    acc = acc + rows_v[tl * _K + k, pl.ds(j2 * _L, _L)]
                acc_v[tl, pl.ds(j2 * _L, _L)] = acc
        pltpu.sync_copy(acc_v, out_hbm.at[pl.ds(wid * tpw + j * tpr, tpr)])
        return carry

    lax.fori_loop(0, tpw // tpr, chunk, 0)


def kernel(x, c_fc_w, c_proj_w, gate_w, expert_bias):
    b, t, d = x.shape
    n_exp, h_dim, _ = c_fc_w.shape
    n_tok = b * t
    x_flat = x.reshape(n_tok, d)
    tpw = n_tok // _NW
    half_pairs = (n_tok // _NC) * _K
    nbh = half_pairs // _B + n_exp
    half_slots = nbh * _B
    nb = _NC * nbh
    npad = _NC * half_slots

    gw_pad = jnp.concatenate(
        [jnp.zeros((1, d), dtype=gate_w.dtype), gate_w], axis=0)
    bias_pad = jnp.concatenate(
        [jnp.full((1,), _NEG, dtype=expert_bias.dtype), expert_bias])
    bias_b = jnp.broadcast_to(bias_pad[:, None], (n_exp, _L))

    lt = min(1024, n_tok)
    logits = pl.pallas_call(
        _logits_kernel,
        grid=(n_tok // lt,),
        in_specs=[
            pl.BlockSpec((lt, d), lambda i: (i, 0)),
            pl.BlockSpec((n_exp, d), lambda i: (0, 0)),
        ],
        out_specs=pl.BlockSpec((n_exp, lt), lambda i: (0, i)),
        out_shape=jax.ShapeDtypeStruct((n_exp, n_tok), jnp.float32),
    )(x_flat, gw_pad)

    lg_blk = logits.reshape(n_exp, _NC, _NS, tpw // _L, _L).transpose(
        1, 2, 3, 0, 4).reshape(_NW, tpw // _L, n_exp, _L)
    zeros_f = jnp.zeros((tpw * n_exp,), jnp.float32)
    zeros_i = jnp.zeros((npad // _NW,), jnp.int32)

    sc_route = functools.partial(
        pl.kernel,
        out_type=(
            jax.ShapeDtypeStruct((n_tok * n_exp,), jnp.float32),
            jax.ShapeDtypeStruct((n_tok * _K // 128, 128), jnp.int32),
            jax.ShapeDtypeStruct((npad,), jnp.int32),
            jax.ShapeDtypeStruct((npad,), jnp.float32),
            jax.ShapeDtypeStruct((nb,), jnp.int32),
            jax.ShapeDtypeStruct((_NW, n_exp), jnp.int32),
        ),
        mesh=plsc.VectorSubcoreMesh(core_axis_name="c", subcore_axis_name="s",
                                    num_cores=_NC, num_subcores=_NS),
        scratch_types=[
            pltpu.VMEM((tpw // _L, n_exp, _L), jnp.float32),
            pltpu.VMEM((n_exp, _L), jnp.float32),
            pltpu.VMEM((tpw * n_exp,), jnp.float32),
            pltpu.VMEM((n_exp * tpw,), jnp.float32),
            pltpu.VMEM((n_exp * tpw,), jnp.float32),
            pltpu.VMEM((tpw * _K,), jnp.int32),
            pltpu.VMEM((tpw * _K // 128, 128), jnp.float32),
            pltpu.VMEM((tpw * _K // 128, 128), jnp.int32),
            pltpu.VMEM((tpw * _K // 128, 128), jnp.int32),
            pltpu.VMEM((n_exp,), jnp.int32),
            pltpu.VMEM((_NS, n_exp), jnp.int32),
            pltpu.VMEM((n_exp,), jnp.int32),
            pltpu.VMEM((n_exp,), jnp.int32),
            pltpu.VMEM((n_exp,), jnp.int32),
            pltpu.VMEM((n_exp,), jnp.int32),
            pltpu.VMEM((nbh + _L,), jnp.int32),
            pltpu.VMEM((npad // _NW,), jnp.float32),
            pltpu.VMEM((npad // _NW,), jnp.int32),
        ],
        compiler_params=pltpu.CompilerParams(needs_layout_passes=False),
    )(functools.partial(_sc_route_body, n_exp=n_exp, tpw=tpw,
                        half_slots=half_slots, nbh=nbh))
    rw_flat, islot, tok_list, w_slot, be_tbl, _ = sc_route(
        lg_blk, bias_b, zeros_f, zeros_i)
    rw = rw_flat.reshape(n_tok, n_exp)

    rows_pw = npad // 128 // _NW
    sc_gather = functools.partial(
        pl.kernel,
        out_type=jax.ShapeDtypeStruct((npad, d), jnp.float32),
        mesh=plsc.VectorSubcoreMesh(core_axis_name="c", subcore_axis_name="s",
                                    num_cores=_NC, num_subcores=_NS),
        scratch_types=[
            pltpu.VMEM((128,), jnp.int32),
            pltpu.VMEM((128, d), jnp.float32),
            pltpu.SemaphoreType.DMA,
        ],
        compiler_params=pltpu.CompilerParams(needs_layout_passes=False),
    )(functools.partial(_sc_gather_body, rows_pw=rows_pw))
    xg = sc_gather(tok_list.reshape(npad // 128, 128), x_flat)

    w2 = w_slot.reshape(nb, _B, 1)
    yg = pl.pallas_call(
        _gemm_kernel,
        grid_spec=pltpu.PrefetchScalarGridSpec(
            num_scalar_prefetch=1,
            grid=(nb,),
            in_specs=[
                pl.BlockSpec((_B, d), lambda i, be: (i, 0)),
                pl.BlockSpec((1, _B, 1), lambda i, be: (i, 0, 0)),
                pl.BlockSpec((1, h_dim, d), lambda i, be: (be[i], 0, 0)),
                pl.BlockSpec((1, d, h_dim), lambda i, be: (be[i], 0, 0)),
            ],
            out_specs=pl.BlockSpec((_B, d), lambda i, be: (i, 0)),
        ),
        out_shape=jax.ShapeDtypeStruct((npad, d), jnp.float32),
        compiler_params=pltpu.CompilerParams(
            dimension_semantics=("arbitrary",)),
    )(be_tbl, xg, w2, c_fc_w, c_proj_w)

    sc_combine = functools.partial(
        pl.kernel,
        out_type=jax.ShapeDtypeStruct((n_tok, d), jnp.float32),
        mesh=plsc.VectorSubcoreMesh(core_axis_name="c", subcore_axis_name="s",
                                    num_cores=_NC, num_subcores=_NS),
        scratch_types=[
            pltpu.VMEM((128,), jnp.int32),
            pltpu.VMEM((128, d), jnp.float32),
            pltpu.VMEM((128 // _K, d), jnp.float32),
            pltpu.SemaphoreType.DMA,
        ],
        compiler_params=pltpu.CompilerParams(needs_layout_passes=False),
    )(functools.partial(_sc_combine_body, d=d, tpw=tpw))
    out = sc_combine(islot, yg)

    return out.reshape(b, t, d), rw


# speedup vs baseline: 4.1334x; 4.1334x over previous
import functools

import jax
import jax.numpy as jnp
from jax import lax
from jax.experimental import pallas as pl
from jax.experimental.pallas import tpu as pltpu
from jax.experimental.pallas import tpu_sc as plsc

_NEG = -1e30
_NC = 2
_NS = 16
_NW = _NC * _NS
_L = 16
_TOPK = 7


def _logits_kernel(x_ref, gw_ref, out_ref):
    out_ref[...] = lax.dot_general(
        gw_ref[...], x_ref[...], (((1,), (1,)), ((), ())),
        preferred_element_type=jnp.float32)


def _sc_route_body(lg_hbm, bias_hbm, zeros_hbm, rw_hbm, lg_v, bias_v, rw_v,
                   *, n_exp, tpw):
    wid = lax.axis_index("s") * _NC + lax.axis_index("c")
    pltpu.sync_copy(lg_hbm.at[wid], lg_v)
    pltpu.sync_copy(bias_hbm, bias_v)
    pltpu.sync_copy(zeros_hbm, rw_v)
    lanes = lax.iota(jnp.int32, _L)

    def group(g, carry):
        m = jnp.full((_L,), _NEG, jnp.float32)
        vs = []
        for r in range(1, n_exp):
            v = lg_v[g, r, :]
            vs.append(v)
            m = jnp.maximum(m, v)
        s = jnp.zeros((_L,), jnp.float32)
        ps = []
        for v in vs:
            p = jnp.exp(v - m)
            ps.append(p)
            s = s + p
        inv = 1.0 / s
        bs = [p * inv + bias_v[r + 1, :] for r, p in enumerate(ps)]
        idxs, vals = [], []
        tot = jnp.zeros((_L,), jnp.float32)
        for _ in range(_TOPK):
            mx = jnp.full((_L,), _NEG, jnp.float32)
            am = jnp.zeros((_L,), jnp.int32)
            for r in range(1, n_exp):
                cand = bs[r - 1]
                ok = cand > mx
                for prev in idxs:
                    ok = jnp.logical_and(ok, prev != r)
                mx = jnp.where(ok, cand, mx)
                am = jnp.where(ok, jnp.full((_L,), r, jnp.int32), am)
            idxs.append(am)
            vals.append(mx)
            tot = tot + mx
        scale = (_TOPK / (_TOPK + 1.0)) / tot
        rowbase = (g * _L + lanes) * n_exp
        plsc.store_scatter(rw_v, [rowbase],
                           jnp.full((_L,), 1.0 / (_TOPK + 1.0), jnp.float32))
        for k in range(_TOPK):
            plsc.store_scatter(rw_v, [rowbase + idxs[k]], vals[k] * scale)
        return carry

    lax.fori_loop(0, tpw // _L, group, 0)
    pltpu.sync_copy(rw_v, rw_hbm.at[pl.ds(wid * tpw * n_exp, tpw * n_exp)])


def _expert0_kernel(x_ref, wfc_ref, wproj_ref, out_ref):
    x = x_ref[...]
    h = lax.dot_general(x, wfc_ref[0], (((1,), (1,)), ((), ())),
                        preferred_element_type=jnp.float32)
    h = jnp.square(jnp.maximum(h, 0.0))
    y = lax.dot_general(h, wproj_ref[0], (((1,), (1,)), ((), ())),
                        preferred_element_type=jnp.float32)
    out_ref[...] = y * (1.0 / (_TOPK + 1.0))


def _expert_kernel(rw_ref, x_ref, wfc_ref, wproj_ref, out0_ref, out_ref):
    e = pl.program_id(1) + 1
    x = x_ref[...]
    h = lax.dot_general(x, wfc_ref[0], (((1,), (1,)), ((), ())),
                        preferred_element_type=jnp.float32)
    h = jnp.square(jnp.maximum(h, 0.0))
    y = lax.dot_general(h, wproj_ref[0], (((1,), (1,)), ((), ())),
                        preferred_element_type=jnp.float32)
    col = lax.broadcasted_iota(jnp.int32, rw_ref.shape, 1)
    w = jnp.sum(rw_ref[...] * (col == e).astype(jnp.float32), axis=1,
                keepdims=True)
    contrib = y * w

    @pl.when(e == 1)
    def _():
        out_ref[...] = out0_ref[...] + contrib

    @pl.when(e != 1)
    def _():
        out_ref[...] += contrib


def kernel(x, c_fc_w, c_proj_w, gate_w, expert_bias):
    b, t, d = x.shape
    n_exp, h_dim, _ = c_fc_w.shape
    n_tok = b * t
    x_flat = x.reshape(n_tok, d)
    tpw = n_tok // _NW

    gw_pad = jnp.concatenate(
        [jnp.zeros((1, d), dtype=gate_w.dtype), gate_w], axis=0)
    bias_pad = jnp.concatenate(
        [jnp.full((1,), _NEG, dtype=expert_bias.dtype), expert_bias])
    bias_b = jnp.broadcast_to(bias_pad[:, None], (n_exp, _L))

    lt = min(1024, n_tok)
    logits = pl.pallas_call(
        _logits_kernel,
        grid=(n_tok // lt,),
        in_specs=[
            pl.BlockSpec((lt, d), lambda i: (i, 0)),
            pl.BlockSpec((n_exp, d), lambda i: (0, 0)),
        ],
        out_specs=pl.BlockSpec((n_exp, lt), lambda i: (0, i)),
        out_shape=jax.ShapeDtypeStruct((n_exp, n_tok), jnp.float32),
    )(x_flat, gw_pad)

    lg_blk = logits.reshape(n_exp, _NW, tpw // _L, _L).transpose(1, 2, 0, 3)
    zeros_rw = jnp.zeros((tpw * n_exp,), jnp.float32)

    sc_route = functools.partial(
        pl.kernel,
        out_type=jax.ShapeDtypeStruct((n_tok * n_exp,), jnp.float32),
        mesh=plsc.VectorSubcoreMesh(core_axis_name="c", subcore_axis_name="s",
                                    num_cores=_NC, num_subcores=_NS),
        scratch_types=[
            pltpu.VMEM((tpw // _L, n_exp, _L), jnp.float32),
            pltpu.VMEM((n_exp, _L), jnp.float32),
            pltpu.VMEM((tpw * n_exp,), jnp.float32),
        ],
        compiler_params=pltpu.CompilerParams(needs_layout_passes=False),
    )(functools.partial(_sc_route_body, n_exp=n_exp, tpw=tpw))
    rw = sc_route(lg_blk, bias_b, zeros_rw).reshape(n_tok, n_exp)

    tt = min(8192, n_tok)
    out0 = pl.pallas_call(
        _expert0_kernel,
        grid=(n_tok // tt,),
        in_specs=[
            pl.BlockSpec((tt, d), lambda i: (i, 0)),
            pl.BlockSpec((1, h_dim, d), lambda i: (0, 0, 0)),
            pl.BlockSpec((1, d, h_dim), lambda i: (0, 0, 0)),
        ],
        out_specs=pl.BlockSpec((tt, d), lambda i: (i, 0)),
        out_shape=jax.ShapeDtypeStruct((n_tok, d), jnp.float32),
    )(x_flat, c_fc_w, c_proj_w)

    out = pl.pallas_call(
        _expert_kernel,
        grid=(n_tok // tt, n_exp - 1),
        in_specs=[
            pl.BlockSpec((tt, n_exp), lambda i, e: (i, 0)),
            pl.BlockSpec((tt, d), lambda i, e: (i, 0)),
            pl.BlockSpec((1, h_dim, d), lambda i, e: (e + 1, 0, 0)),
            pl.BlockSpec((1, d, h_dim), lambda i, e: (e + 1, 0, 0)),
            pl.BlockSpec((tt, d), lambda i, e: (i, 0)),
        ],
        out_specs=pl.BlockSpec((tt, d), lambda i, e: (i, 0)),
        out_shape=jax.ShapeDtypeStruct((n_tok, d), jnp.float32),
        compiler_params=pltpu.CompilerParams(
            dimension_semantics=("parallel", "arbitrary")),
    )(rw, x_flat, c_fc_w, c_proj_w, out0)

    return out.reshape(b, t, d), rw


# speedup vs baseline: 4.1575x; 1.0058x over previous
import functools

import jax
import jax.numpy as jnp
from jax import lax
from jax.experimental import pallas as pl
from jax.experimental.pallas import tpu as pltpu
from jax.experimental.pallas import tpu_sc as plsc

_NEG = -1e30
_NC = 2
_NS = 16
_NW = _NC * _NS
_L = 16
_TOPK = 7


def _logits_kernel(x_ref, gw_ref, out_ref):
    out_ref[...] = lax.dot_general(
        gw_ref[...], x_ref[...], (((1,), (1,)), ((), ())),
        preferred_element_type=jnp.float32)


def _sc_route_body(lg_hbm, bias_hbm, zeros_hbm, rw_hbm, lg_v, bias_v, rw_v,
                   *, n_exp, tpw):
    wid = lax.axis_index("s") * _NC + lax.axis_index("c")
    pltpu.sync_copy(lg_hbm.at[wid], lg_v)
    pltpu.sync_copy(bias_hbm, bias_v)
    pltpu.sync_copy(zeros_hbm, rw_v)
    lanes = lax.iota(jnp.int32, _L)

    def group(g, carry):
        m = jnp.full((_L,), _NEG, jnp.float32)
        vs = []
        for r in range(1, n_exp):
            v = lg_v[g, r, :]
            vs.append(v)
            m = jnp.maximum(m, v)
        s = jnp.zeros((_L,), jnp.float32)
        ps = []
        for v in vs:
            p = jnp.exp(v - m)
            ps.append(p)
            s = s + p
        inv = 1.0 / s
        bs = [p * inv + bias_v[r + 1, :] for r, p in enumerate(ps)]
        idxs, vals = [], []
        tot = jnp.zeros((_L,), jnp.float32)
        for _ in range(_TOPK):
            mx = jnp.full((_L,), _NEG, jnp.float32)
            am = jnp.zeros((_L,), jnp.int32)
            for r in range(1, n_exp):
                cand = bs[r - 1]
                ok = cand > mx
                for prev in idxs:
                    ok = jnp.logical_and(ok, prev != r)
                mx = jnp.where(ok, cand, mx)
                am = jnp.where(ok, jnp.full((_L,), r, jnp.int32), am)
            idxs.append(am)
            vals.append(mx)
            tot = tot + mx
        scale = (_TOPK / (_TOPK + 1.0)) / tot
        rowbase = (g * _L + lanes) * n_exp
        plsc.store_scatter(rw_v, [rowbase],
                           jnp.full((_L,), 1.0 / (_TOPK + 1.0), jnp.float32))
        for k in range(_TOPK):
            plsc.store_scatter(rw_v, [rowbase + idxs[k]], vals[k] * scale)
        return carry

    lax.fori_loop(0, tpw // _L, group, 0)
    pltpu.sync_copy(rw_v, rw_hbm.at[pl.ds(wid * tpw * n_exp, tpw * n_exp)])


def _expert01_kernel(x_ref, wfc_ref, wproj_ref, out0_ref, y1_ref):
    x = x_ref[...]
    for e, ref, scale in ((0, out0_ref, 1.0 / (_TOPK + 1.0)), (1, y1_ref, 1.0)):
        h = lax.dot_general(x, wfc_ref[e], (((1,), (1,)), ((), ())),
                            preferred_element_type=jnp.float32)
        h = jnp.square(jnp.maximum(h, 0.0))
        y = lax.dot_general(h, wproj_ref[e], (((1,), (1,)), ((), ())),
                            preferred_element_type=jnp.float32)
        ref[...] = y * scale


def _expert_kernel(rw_ref, x_ref, wfc_ref, wproj_ref, out0_ref, y1_ref,
                   out_ref):
    e = pl.program_id(1) + 2
    x = x_ref[...]
    h = lax.dot_general(x, wfc_ref[0], (((1,), (1,)), ((), ())),
                        preferred_element_type=jnp.float32)
    h = jnp.square(jnp.maximum(h, 0.0))
    y = lax.dot_general(h, wproj_ref[0], (((1,), (1,)), ((), ())),
                        preferred_element_type=jnp.float32)
    col = lax.broadcasted_iota(jnp.int32, rw_ref.shape, 1)
    rw = rw_ref[...]
    w = jnp.sum(rw * (col == e).astype(jnp.float32), axis=1, keepdims=True)
    contrib = y * w

    @pl.when(e == 2)
    def _():
        w1 = jnp.sum(rw * (col == 1).astype(jnp.float32), axis=1,
                     keepdims=True)
        out_ref[...] = out0_ref[...] + y1_ref[...] * w1 + contrib

    @pl.when(e != 2)
    def _():
        out_ref[...] += contrib


def kernel(x, c_fc_w, c_proj_w, gate_w, expert_bias):
    b, t, d = x.shape
    n_exp, h_dim, _ = c_fc_w.shape
    n_tok = b * t
    x_flat = x.reshape(n_tok, d)
    tpw = n_tok // _NW

    gw_pad = jnp.concatenate(
        [jnp.zeros((1, d), dtype=gate_w.dtype), gate_w], axis=0)
    bias_pad = jnp.concatenate(
        [jnp.full((1,), _NEG, dtype=expert_bias.dtype), expert_bias])
    bias_b = jnp.broadcast_to(bias_pad[:, None], (n_exp, _L))

    lt = min(1024, n_tok)
    logits = pl.pallas_call(
        _logits_kernel,
        grid=(n_tok // lt,),
        in_specs=[
            pl.BlockSpec((lt, d), lambda i: (i, 0)),
            pl.BlockSpec((n_exp, d), lambda i: (0, 0)),
        ],
        out_specs=pl.BlockSpec((n_exp, lt), lambda i: (0, i)),
        out_shape=jax.ShapeDtypeStruct((n_exp, n_tok), jnp.float32),
    )(x_flat, gw_pad)

    lg_blk = logits.reshape(n_exp, _NW, tpw // _L, _L).transpose(1, 2, 0, 3)
    zeros_rw = jnp.zeros((tpw * n_exp,), jnp.float32)

    sc_route = functools.partial(
        pl.kernel,
        out_type=jax.ShapeDtypeStruct((n_tok * n_exp,), jnp.float32),
        mesh=plsc.VectorSubcoreMesh(core_axis_name="c", subcore_axis_name="s",
                                    num_cores=_NC, num_subcores=_NS),
        scratch_types=[
            pltpu.VMEM((tpw // _L, n_exp, _L), jnp.float32),
            pltpu.VMEM((n_exp, _L), jnp.float32),
            pltpu.VMEM((tpw * n_exp,), jnp.float32),
        ],
        compiler_params=pltpu.CompilerParams(needs_layout_passes=False),
    )(functools.partial(_sc_route_body, n_exp=n_exp, tpw=tpw))
    rw = sc_route(lg_blk, bias_b, zeros_rw).reshape(n_tok, n_exp)

    tt = min(8192, n_tok)
    out0, y1 = pl.pallas_call(
        _expert01_kernel,
        grid=(n_tok // tt,),
        in_specs=[
            pl.BlockSpec((tt, d), lambda i: (i, 0)),
            pl.BlockSpec((2, h_dim, d), lambda i: (0, 0, 0)),
            pl.BlockSpec((2, d, h_dim), lambda i: (0, 0, 0)),
        ],
        out_specs=[
            pl.BlockSpec((tt, d), lambda i: (i, 0)),
            pl.BlockSpec((tt, d), lambda i: (i, 0)),
        ],
        out_shape=[
            jax.ShapeDtypeStruct((n_tok, d), jnp.float32),
            jax.ShapeDtypeStruct((n_tok, d), jnp.float32),
        ],
    )(x_flat, c_fc_w, c_proj_w)

    out = pl.pallas_call(
        _expert_kernel,
        grid=(n_tok // tt, n_exp - 2),
        in_specs=[
            pl.BlockSpec((tt, n_exp), lambda i, e: (i, 0)),
            pl.BlockSpec((tt, d), lambda i, e: (i, 0)),
            pl.BlockSpec((1, h_dim, d), lambda i, e: (e + 2, 0, 0)),
            pl.BlockSpec((1, d, h_dim), lambda i, e: (e + 2, 0, 0)),
            pl.BlockSpec((tt, d), lambda i, e: (i, 0)),
            pl.BlockSpec((tt, d), lambda i, e: (i, 0)),
        ],
        out_specs=pl.BlockSpec((tt, d), lambda i, e: (i, 0)),
        out_shape=jax.ShapeDtypeStruct((n_tok, d), jnp.float32),
        compiler_params=pltpu.CompilerParams(
            dimension_semantics=("parallel", "arbitrary")),
    )(rw, x_flat, c_fc_w, c_proj_w, out0, y1)

    return out.reshape(b, t, d), rw


# speedup vs baseline: 4.2390x; 1.0196x over previous
import functools

import jax
import jax.numpy as jnp
from jax import lax
from jax.experimental import pallas as pl
from jax.experimental.pallas import tpu as pltpu
from jax.experimental.pallas import tpu_sc as plsc

_NEG = -1e30
_NC = 2
_NS = 16
_NW = _NC * _NS
_L = 16
_TOPK = 7


def _logits_kernel(x_ref, gw_ref, out_ref):
    out_ref[...] = lax.dot_general(
        gw_ref[...], x_ref[...], (((1,), (1,)), ((), ())),
        preferred_element_type=jnp.float32)


def _sc_route_body(lg_hbm, bias_hbm, zeros_hbm, rw_hbm, lg_v, bias_v, rw_v,
                   *, n_exp, tpw):
    wid = lax.axis_index("s") * _NC + lax.axis_index("c")
    pltpu.sync_copy(lg_hbm.at[:, pl.ds(wid * tpw, tpw)], lg_v)
    pltpu.sync_copy(bias_hbm, bias_v)
    pltpu.sync_copy(zeros_hbm, rw_v)
    lanes = lax.iota(jnp.int32, _L)

    def group(g, carry):
        m = jnp.full((_L,), _NEG, jnp.float32)
        vs = []
        for r in range(1, n_exp):
            v = lg_v[r, pl.ds(g * _L, _L)]
            vs.append(v)
            m = jnp.maximum(m, v)
        s = jnp.zeros((_L,), jnp.float32)
        ps = []
        for v in vs:
            p = jnp.exp(v - m)
            ps.append(p)
            s = s + p
        inv = 1.0 / s
        bs = [p * inv + bias_v[r + 1, :] for r, p in enumerate(ps)]
        idxs, vals = [], []
        tot = jnp.zeros((_L,), jnp.float32)
        for _ in range(_TOPK):
            mx = jnp.full((_L,), _NEG, jnp.float32)
            am = jnp.zeros((_L,), jnp.int32)
            for r in range(1, n_exp):
                cand = bs[r - 1]
                ok = cand > mx
                for prev in idxs:
                    ok = jnp.logical_and(ok, prev != r)
                mx = jnp.where(ok, cand, mx)
                am = jnp.where(ok, jnp.full((_L,), r, jnp.int32), am)
            idxs.append(am)
            vals.append(mx)
            tot = tot + mx
        scale = (_TOPK / (_TOPK + 1.0)) / tot
        rowbase = (g * _L + lanes) * n_exp
        plsc.store_scatter(rw_v, [rowbase],
                           jnp.full((_L,), 1.0 / (_TOPK + 1.0), jnp.float32))
        for k in range(_TOPK):
            plsc.store_scatter(rw_v, [rowbase + idxs[k]], vals[k] * scale)
        return carry

    lax.fori_loop(0, tpw // _L, group, 0)
    pltpu.sync_copy(rw_v, rw_hbm.at[pl.ds(wid * tpw * n_exp, tpw * n_exp)])


def _expert01_kernel(x_ref, wfc_ref, wproj_ref, out0_ref, y1_ref):
    x = x_ref[...]
    for e, ref, scale in ((0, out0_ref, 1.0 / (_TOPK + 1.0)), (1, y1_ref, 1.0)):
        h = lax.dot_general(x, wfc_ref[e], (((1,), (1,)), ((), ())),
                            preferred_element_type=jnp.float32)
        h = jnp.square(jnp.maximum(h, 0.0))
        y = lax.dot_general(h, wproj_ref[e], (((1,), (1,)), ((), ())),
                            preferred_element_type=jnp.float32)
        ref[...] = y * scale


def _expert_kernel(rw_ref, x_ref, wfc_ref, wproj_ref, out0_ref, y1_ref,
                   out_ref):
    e = pl.program_id(1) + 2
    x = x_ref[...]
    h = lax.dot_general(x, wfc_ref[0], (((1,), (1,)), ((), ())),
                        preferred_element_type=jnp.float32)
    h = jnp.square(jnp.maximum(h, 0.0))
    y = lax.dot_general(h, wproj_ref[0], (((1,), (1,)), ((), ())),
                        preferred_element_type=jnp.float32)
    col = lax.broadcasted_iota(jnp.int32, rw_ref.shape, 1)
    rw = rw_ref[...]
    w = jnp.sum(rw * (col == e).astype(jnp.float32), axis=1, keepdims=True)
    contrib = y * w

    @pl.when(e == 2)
    def _():
        w1 = jnp.sum(rw * (col == 1).astype(jnp.float32), axis=1,
                     keepdims=True)
        out_ref[...] = out0_ref[...] + y1_ref[...] * w1 + contrib

    @pl.when(e != 2)
    def _():
        out_ref[...] += contrib


def kernel(x, c_fc_w, c_proj_w, gate_w, expert_bias):
    b, t, d = x.shape
    n_exp, h_dim, _ = c_fc_w.shape
    n_tok = b * t
    x_flat = x.reshape(n_tok, d)
    tpw = n_tok // _NW

    gw_pad = jnp.concatenate(
        [jnp.zeros((1, d), dtype=gate_w.dtype), gate_w], axis=0)
    bias_pad = jnp.concatenate(
        [jnp.full((1,), _NEG, dtype=expert_bias.dtype), expert_bias])
    bias_b = jnp.broadcast_to(bias_pad[:, None], (n_exp, _L))

    lt = min(1024, n_tok)
    logits = pl.pallas_call(
        _logits_kernel,
        grid=(n_tok // lt,),
        in_specs=[
            pl.BlockSpec((lt, d), lambda i: (i, 0)),
            pl.BlockSpec((n_exp, d), lambda i: (0, 0)),
        ],
        out_specs=pl.BlockSpec((n_exp, lt), lambda i: (0, i)),
        out_shape=jax.ShapeDtypeStruct((n_exp, n_tok), jnp.float32),
    )(x_flat, gw_pad)

    zeros_rw = jnp.zeros((tpw * n_exp,), jnp.float32)

    sc_route = functools.partial(
        pl.kernel,
        out_type=jax.ShapeDtypeStruct((n_tok * n_exp,), jnp.float32),
        mesh=plsc.VectorSubcoreMesh(core_axis_name="c", subcore_axis_name="s",
                                    num_cores=_NC, num_subcores=_NS),
        scratch_types=[
            pltpu.VMEM((n_exp, tpw), jnp.float32),
            pltpu.VMEM((n_exp, _L), jnp.float32),
            pltpu.VMEM((tpw * n_exp,), jnp.float32),
        ],
        compiler_params=pltpu.CompilerParams(needs_layout_passes=False),
    )(functools.partial(_sc_route_body, n_exp=n_exp, tpw=tpw))
    rw = sc_route(logits, bias_b, zeros_rw).reshape(n_tok, n_exp)

    tt = min(8192, n_tok)
    out0, y1 = pl.pallas_call(
        _expert01_kernel,
        grid=(n_tok // tt,),
        in_specs=[
            pl.BlockSpec((tt, d), lambda i: (i, 0)),
            pl.BlockSpec((2, h_dim, d), lambda i: (0, 0, 0)),
            pl.BlockSpec((2, d, h_dim), lambda i: (0, 0, 0)),
        ],
        out_specs=[
            pl.BlockSpec((tt, d), lambda i: (i, 0)),
            pl.BlockSpec((tt, d), lambda i: (i, 0)),
        ],
        out_shape=[
            jax.ShapeDtypeStruct((n_tok, d), jnp.float32),
            jax.ShapeDtypeStruct((n_tok, d), jnp.float32),
        ],
    )(x_flat, c_fc_w, c_proj_w)

    out = pl.pallas_call(
        _expert_kernel,
        grid=(n_tok // tt, n_exp - 2),
        in_specs=[
            pl.BlockSpec((tt, n_exp), lambda i, e: (i, 0)),
            pl.BlockSpec((tt, d), lambda i, e: (i, 0)),
            pl.BlockSpec((1, h_dim, d), lambda i, e: (e + 2, 0, 0)),
            pl.BlockSpec((1, d, h_dim), lambda i, e: (e + 2, 0, 0)),
            pl.BlockSpec((tt, d), lambda i, e: (i, 0)),
            pl.BlockSpec((tt, d), lambda i, e: (i, 0)),
        ],
        out_specs=pl.BlockSpec((tt, d), lambda i, e: (i, 0)),
        out_shape=jax.ShapeDtypeStruct((n_tok, d), jnp.float32),
        compiler_params=pltpu.CompilerParams(
            dimension_semantics=("parallel", "arbitrary")),
    )(rw, x_flat, c_fc_w, c_proj_w, out0, y1)

    return out.reshape(b, t, d), rw
